# Initial kernel scaffold; baseline (speedup 1.0000x reference)
import jax
import jax.numpy as jnp
from jax.experimental import pallas as pl


def kernel(x, edge_index, W1, b1, W2, b2, Wh, bh):
    raise NotImplementedError("write your pallas kernel here")


# speedup vs baseline: 11.1218x; 11.1218x over previous
import functools

import jax
import jax.numpy as jnp
from jax import lax
from jax.experimental import pallas as pl
from jax.experimental.pallas import tpu as pltpu
from jax.experimental.pallas import tpu_sc as plsc

N = 10000
E = 320000
IN_CH = 128
HIDDEN = 128
HID2 = 64

NC = 2
NS = 16
NW = NC * NS

N_PAD = 10240
CHUNK = 128
EDGES_PER_W = 10112
N_CHUNKS = EDGES_PER_W // CHUNK
E_PAD = NW * EDGES_PER_W
ROWS_PER_TILE = N_PAD // NS

_mesh = plsc.VectorSubcoreMesh(core_axis_name="c", subcore_axis_name="s")


@functools.partial(
    pl.kernel,
    mesh=_mesh,
    out_type=jax.ShapeDtypeStruct((NW, N_PAD), jnp.float32),
    scratch_types=[
        pltpu.VMEM((CHUNK,), jnp.int32),
        pltpu.VMEM((N_PAD,), jnp.float32),
    ],
    compiler_params=pltpu.CompilerParams(needs_layout_passes=False),
)
def _deg_kernel(dst_hbm, out_hbm, idx_v, dacc):
    c = lax.axis_index("c")
    s = lax.axis_index("s")
    wid = s * NC + c
    zero16 = jnp.zeros((16,), jnp.float32)
    one16 = jnp.ones((16,), jnp.float32)

    def zero_body(i, carry):
        dacc[pl.ds(i * 16, 16)] = zero16
        return carry

    lax.fori_loop(0, N_PAD // 16, zero_body, 0)

    base = wid * EDGES_PER_W

    def chunk_body(i, carry):
        pltpu.sync_copy(dst_hbm.at[pl.ds(base + i * CHUNK, CHUNK)], idx_v)

        def inner(j, c2):
            idx = idx_v[pl.ds(j * 16, 16)]
            plsc.addupdate_scatter(dacc, [idx], one16)
            return c2

        lax.fori_loop(0, CHUNK // 16, inner, 0)
        return carry

    lax.fori_loop(0, N_CHUNKS, chunk_body, 0)
    pltpu.sync_copy(dacc, out_hbm.at[wid])


def _make_agg(D):
    @functools.partial(
        pl.kernel,
        mesh=_mesh,
        out_type=jax.ShapeDtypeStruct((NC, N_PAD, D), jnp.float32),
        scratch_types=[
            pltpu.VMEM((CHUNK,), jnp.int32),
            pltpu.VMEM((CHUNK,), jnp.int32),
            pltpu.VMEM((CHUNK, D), jnp.float32),
            pltpu.VMEM_SHARED((N_PAD, D), jnp.float32),
            pltpu.SemaphoreType.DMA,
        ],
        compiler_params=pltpu.CompilerParams(
            use_tc_tiling_on_sc=(D % 128 == 0)),
    )
    def agg(src_hbm, dst_hbm, g_hbm, zeros_hbm, out_hbm,
            src_v, dst_v, rows_v, acc, sem):
        c = lax.axis_index("c")
        s = lax.axis_index("s")
        pltpu.sync_copy(zeros_hbm, rows_v)
        row0 = s * ROWS_PER_TILE
        for k in range(ROWS_PER_TILE // CHUNK):
            pltpu.sync_copy(rows_v, acc.at[pl.ds(row0 + k * CHUNK, CHUNK)])
        plsc.subcore_barrier()

        wid = c * NS + s
        base = wid * EDGES_PER_W

        def chunk_body(i, carry):
            off = base + i * CHUNK
            pltpu.sync_copy(src_hbm.at[pl.ds(off, CHUNK)], src_v)
            pltpu.sync_copy(dst_hbm.at[pl.ds(off, CHUNK)], dst_v)
            pltpu.async_copy(g_hbm.at[src_v], rows_v, sem).wait()
            pltpu.sync_copy(rows_v, acc.at[dst_v], add=True)
            return carry

        lax.fori_loop(0, N_CHUNKS, chunk_body, 0)
        plsc.subcore_barrier()

        for k in range(ROWS_PER_TILE // CHUNK):
            r = row0 + k * CHUNK
            pltpu.sync_copy(acc.at[pl.ds(r, CHUNK)], rows_v)
            pltpu.sync_copy(rows_v, out_hbm.at[c, pl.ds(r, CHUNK)])

    return agg


_agg_h = _make_agg(HIDDEN)
_agg_h2 = _make_agg(HID2)


BLK = 256


def _t1_body(x_ref, w1_ref, degs_ref, g_ref, dinv_ref):
    deg = jnp.sum(degs_ref[...], axis=0) + 1.0
    dinv = lax.rsqrt(deg)
    h = lax.dot_general(x_ref[...], w1_ref[...], (((1,), (1,)), ((), ())),
                        preferred_element_type=jnp.float32)
    g_ref[...] = h * dinv[:, None]
    dinv_ref[...] = dinv


_t1 = pl.pallas_call(
    _t1_body,
    grid=(N_PAD // BLK,),
    in_specs=[
        pl.BlockSpec((BLK, IN_CH), lambda i: (i, 0)),
        pl.BlockSpec((HIDDEN, IN_CH), lambda i: (0, 0)),
        pl.BlockSpec((NW, BLK), lambda i: (0, i)),
    ],
    out_specs=[
        pl.BlockSpec((BLK, HIDDEN), lambda i: (i, 0)),
        pl.BlockSpec((BLK,), lambda i: (i,)),
    ],
    out_shape=[
        jax.ShapeDtypeStruct((N_PAD, HIDDEN), jnp.float32),
        jax.ShapeDtypeStruct((N_PAD,), jnp.float32),
    ],
)


def _t2_body(a0_ref, a1_ref, g1_ref, dinv_ref, b1_ref, w2_ref, g2_ref):
    dinv = dinv_ref[...]
    h1 = (a0_ref[...] + a1_ref[...] + g1_ref[...]) * dinv[:, None]
    h1 = jnp.maximum(h1 + b1_ref[...][None, :], 0.0)
    h2 = lax.dot_general(h1, w2_ref[...], (((1,), (1,)), ((), ())),
                         preferred_element_type=jnp.float32)
    g2_ref[...] = h2 * dinv[:, None]


_t2 = pl.pallas_call(
    _t2_body,
    grid=(N_PAD // BLK,),
    in_specs=[
        pl.BlockSpec((BLK, HIDDEN), lambda i: (i, 0)),
        pl.BlockSpec((BLK, HIDDEN), lambda i: (i, 0)),
        pl.BlockSpec((BLK, HIDDEN), lambda i: (i, 0)),
        pl.BlockSpec((BLK,), lambda i: (i,)),
        pl.BlockSpec((HIDDEN,), lambda i: (0,)),
        pl.BlockSpec((HID2, HIDDEN), lambda i: (0, 0)),
    ],
    out_specs=pl.BlockSpec((BLK, HID2), lambda i: (i, 0)),
    out_shape=jax.ShapeDtypeStruct((N_PAD, HID2), jnp.float32),
)


def _t3_body(a0_ref, a1_ref, g2_ref, dinv_ref, b2_ref, wh_ref, bh_ref, out_ref):
    dinv = dinv_ref[...]
    h2 = (a0_ref[...] + a1_ref[...] + g2_ref[...]) * dinv[:, None]
    h2 = jnp.maximum(h2 + b2_ref[...][None, :], 0.0)
    out_ref[...] = jnp.sum(h2 * wh_ref[...][None, :], axis=1) + bh_ref[0]


_t3 = pl.pallas_call(
    _t3_body,
    grid=(N_PAD // BLK,),
    in_specs=[
        pl.BlockSpec((BLK, HID2), lambda i: (i, 0)),
        pl.BlockSpec((BLK, HID2), lambda i: (i, 0)),
        pl.BlockSpec((BLK, HID2), lambda i: (i, 0)),
        pl.BlockSpec((BLK,), lambda i: (i,)),
        pl.BlockSpec((HID2,), lambda i: (0,)),
        pl.BlockSpec((HID2,), lambda i: (0,)),
        pl.BlockSpec((1,), lambda i: (0,)),
    ],
    out_specs=pl.BlockSpec((BLK,), lambda i: (i,)),
    out_shape=jax.ShapeDtypeStruct((N_PAD,), jnp.float32),
)


def kernel(x, edge_index, W1, b1, W2, b2, Wh, bh):
    src = edge_index[0].astype(jnp.int32)
    dst = edge_index[1].astype(jnp.int32)
    dummy = jnp.full((E_PAD - E,), N_PAD - 1, jnp.int32)
    src_p = jnp.concatenate([src, dummy])
    dst_p = jnp.concatenate([dst, dummy])
    x_p = jnp.concatenate(
        [x, jnp.zeros((N_PAD - N, IN_CH), x.dtype)], axis=0)

    degs = _deg_kernel(dst_p)
    g1, dinv = _t1(x_p, W1, degs)
    z_h = jnp.zeros((CHUNK, HIDDEN), jnp.float32)
    agg1 = _agg_h(src_p, dst_p, g1, z_h)
    g2 = _t2(agg1[0], agg1[1], g1, dinv, b1, W2)
    z_h2 = jnp.zeros((CHUNK, HID2), jnp.float32)
    agg2 = _agg_h2(src_p, dst_p, g2, z_h2)
    logits = _t3(agg2[0], agg2[1], g2, dinv, b2, Wh.reshape(-1), bh)
    return logits[:N]


# speedup vs baseline: 12.0341x; 1.0820x over previous
import functools

import jax
import jax.numpy as jnp
from jax import lax
from jax.experimental import pallas as pl
from jax.experimental.pallas import tpu as pltpu
from jax.experimental.pallas import tpu_sc as plsc

N = 10000
E = 320000
IN_CH = 128
HIDDEN = 128
HID2 = 64

NC = 2
NS = 16
NW = NC * NS

N_PAD = 10240
CHUNK = 128
N_CHUNKS = 80
EDGES_PER_W = N_CHUNKS * CHUNK
E_PAD = NW * EDGES_PER_W
ROWS_PER_TILE = N_PAD // NS

_mesh = plsc.VectorSubcoreMesh(core_axis_name="c", subcore_axis_name="s")


@functools.partial(
    pl.kernel,
    mesh=_mesh,
    out_type=jax.ShapeDtypeStruct((NW, N_PAD), jnp.float32),
    scratch_types=[
        pltpu.VMEM((N_CHUNKS, CHUNK), jnp.int32),
        pltpu.VMEM((N_PAD,), jnp.float32),
    ],
    compiler_params=pltpu.CompilerParams(needs_layout_passes=False),
)
def _deg_kernel(dst_hbm, out_hbm, idx_v, dacc):
    c = lax.axis_index("c")
    s = lax.axis_index("s")
    wid = s * NC + c
    zero16 = jnp.zeros((16,), jnp.float32)
    one16 = jnp.ones((16,), jnp.float32)

    def zero_body(i, carry):
        dacc[pl.ds(i * 16, 16)] = zero16
        return carry

    lax.fori_loop(0, N_PAD // 16, zero_body, 0)

    pltpu.sync_copy(dst_hbm.at[pl.ds(wid * N_CHUNKS, N_CHUNKS)], idx_v)

    def inner(j, c2):
        idx = idx_v[j // (CHUNK // 16), pl.ds((j % (CHUNK // 16)) * 16, 16)]
        plsc.addupdate_scatter(dacc, [idx], one16)
        return c2

    lax.fori_loop(0, EDGES_PER_W // 16, inner, 0)
    pltpu.sync_copy(dacc, out_hbm.at[wid])


def _make_agg(D):
    seg = 40 if D >= 128 else N_CHUNKS
    nseg = N_CHUNKS // seg

    @functools.partial(
        pl.kernel,
        mesh=_mesh,
        out_type=jax.ShapeDtypeStruct((NC, N_PAD, D), jnp.float32),
        scratch_types=[
            pltpu.VMEM((seg, CHUNK), jnp.int32),
            pltpu.VMEM((N_CHUNKS, CHUNK), jnp.int32),
            pltpu.VMEM((CHUNK, D), jnp.float32),
            pltpu.VMEM((CHUNK, D), jnp.float32),
            pltpu.VMEM_SHARED((N_PAD, D), jnp.float32),
            pltpu.SemaphoreType.DMA,
            pltpu.SemaphoreType.DMA,
        ],
        compiler_params=pltpu.CompilerParams(
            use_tc_tiling_on_sc=(D % 128 == 0)),
    )
    def agg(src_hbm, dst_hbm, g_hbm, zeros_hbm, out_hbm,
            src_v, dst_v, rows_a, rows_b, acc, sem_a, sem_b):
        c = lax.axis_index("c")
        s = lax.axis_index("s")
        pltpu.sync_copy(zeros_hbm, rows_a)
        row0 = s * ROWS_PER_TILE
        for k in range(ROWS_PER_TILE // CHUNK):
            pltpu.sync_copy(rows_a, acc.at[pl.ds(row0 + k * CHUNK, CHUNK)])

        wid = c * NS + s
        pltpu.sync_copy(dst_hbm.at[pl.ds(wid * N_CHUNKS, N_CHUNKS)], dst_v)
        plsc.subcore_barrier()

        for sg in range(nseg):
            cbase = sg * seg
            pltpu.sync_copy(
                src_hbm.at[pl.ds(wid * N_CHUNKS + cbase, seg)], src_v)
            pltpu.async_copy(g_hbm.at[src_v.at[0]], rows_a, sem_a)

            def pipe_body(g, carry):
                i0 = 2 * g
                pltpu.async_copy(g_hbm.at[src_v.at[i0 + 1]], rows_b, sem_b)
                pltpu.make_async_copy(
                    g_hbm.at[src_v.at[i0]], rows_a, sem_a).wait()
                pltpu.sync_copy(rows_a, acc.at[dst_v.at[cbase + i0]],
                                add=True)

                @pl.when(g < seg // 2 - 1)
                def _():
                    pltpu.async_copy(g_hbm.at[src_v.at[i0 + 2]], rows_a,
                                     sem_a)

                pltpu.make_async_copy(
                    g_hbm.at[src_v.at[i0 + 1]], rows_b, sem_b).wait()
                pltpu.sync_copy(rows_b, acc.at[dst_v.at[cbase + i0 + 1]],
                                add=True)
                return carry

            lax.fori_loop(0, seg // 2, pipe_body, 0)

        plsc.subcore_barrier()
        for k in range(ROWS_PER_TILE // CHUNK):
            r = row0 + k * CHUNK
            pltpu.sync_copy(acc.at[pl.ds(r, CHUNK)], rows_a)
            pltpu.sync_copy(rows_a, out_hbm.at[c, pl.ds(r, CHUNK)])

    return agg


_agg_h = _make_agg(HIDDEN)
_agg_h2 = _make_agg(HID2)


BLK = 256


def _t1_body(x_ref, w1_ref, degs_ref, g_ref, dinv_ref):
    deg = jnp.sum(degs_ref[...], axis=0) + 1.0
    dinv = lax.rsqrt(deg)
    h = lax.dot_general(x_ref[...], w1_ref[...], (((1,), (1,)), ((), ())),
                        preferred_element_type=jnp.float32)
    g_ref[...] = h * dinv[:, None]
    dinv_ref[...] = dinv


_t1 = pl.pallas_call(
    _t1_body,
    grid=(N_PAD // BLK,),
    in_specs=[
        pl.BlockSpec((BLK, IN_CH), lambda i: (i, 0)),
        pl.BlockSpec((HIDDEN, IN_CH), lambda i: (0, 0)),
        pl.BlockSpec((NW, BLK), lambda i: (0, i)),
    ],
    out_specs=[
        pl.BlockSpec((BLK, HIDDEN), lambda i: (i, 0)),
        pl.BlockSpec((BLK,), lambda i: (i,)),
    ],
    out_shape=[
        jax.ShapeDtypeStruct((N_PAD, HIDDEN), jnp.float32),
        jax.ShapeDtypeStruct((N_PAD,), jnp.float32),
    ],
)


def _t2_body(a0_ref, a1_ref, g1_ref, dinv_ref, b1_ref, w2_ref, g2_ref):
    dinv = dinv_ref[...]
    h1 = (a0_ref[...] + a1_ref[...] + g1_ref[...]) * dinv[:, None]
    h1 = jnp.maximum(h1 + b1_ref[...][None, :], 0.0)
    h2 = lax.dot_general(h1, w2_ref[...], (((1,), (1,)), ((), ())),
                         preferred_element_type=jnp.float32)
    g2_ref[...] = h2 * dinv[:, None]


_t2 = pl.pallas_call(
    _t2_body,
    grid=(N_PAD // BLK,),
    in_specs=[
        pl.BlockSpec((BLK, HIDDEN), lambda i: (i, 0)),
        pl.BlockSpec((BLK, HIDDEN), lambda i: (i, 0)),
        pl.BlockSpec((BLK, HIDDEN), lambda i: (i, 0)),
        pl.BlockSpec((BLK,), lambda i: (i,)),
        pl.BlockSpec((HIDDEN,), lambda i: (0,)),
        pl.BlockSpec((HID2, HIDDEN), lambda i: (0, 0)),
    ],
    out_specs=pl.BlockSpec((BLK, HID2), lambda i: (i, 0)),
    out_shape=jax.ShapeDtypeStruct((N_PAD, HID2), jnp.float32),
)


def _t3_body(a0_ref, a1_ref, g2_ref, dinv_ref, b2_ref, wh_ref, bh_ref, out_ref):
    dinv = dinv_ref[...]
    h2 = (a0_ref[...] + a1_ref[...] + g2_ref[...]) * dinv[:, None]
    h2 = jnp.maximum(h2 + b2_ref[...][None, :], 0.0)
    out_ref[...] = jnp.sum(h2 * wh_ref[...][None, :], axis=1) + bh_ref[0]


_t3 = pl.pallas_call(
    _t3_body,
    grid=(N_PAD // BLK,),
    in_specs=[
        pl.BlockSpec((BLK, HID2), lambda i: (i, 0)),
        pl.BlockSpec((BLK, HID2), lambda i: (i, 0)),
        pl.BlockSpec((BLK, HID2), lambda i: (i, 0)),
        pl.BlockSpec((BLK,), lambda i: (i,)),
        pl.BlockSpec((HID2,), lambda i: (0,)),
        pl.BlockSpec((HID2,), lambda i: (0,)),
        pl.BlockSpec((1,), lambda i: (0,)),
    ],
    out_specs=pl.BlockSpec((BLK,), lambda i: (i,)),
    out_shape=jax.ShapeDtypeStruct((N_PAD,), jnp.float32),
)


def kernel(x, edge_index, W1, b1, W2, b2, Wh, bh):
    src = edge_index[0].astype(jnp.int32)
    dst = edge_index[1].astype(jnp.int32)
    dummy = jnp.full((E_PAD - E,), N_PAD - 1, jnp.int32)
    src_p = jnp.concatenate([src, dummy]).reshape(NW * N_CHUNKS, CHUNK)
    dst_p = jnp.concatenate([dst, dummy]).reshape(NW * N_CHUNKS, CHUNK)
    x_p = jnp.concatenate(
        [x, jnp.zeros((N_PAD - N, IN_CH), x.dtype)], axis=0)

    degs = _deg_kernel(dst_p)
    g1, dinv = _t1(x_p, W1, degs)
    z_h = jnp.zeros((CHUNK, HIDDEN), jnp.float32)
    agg1 = _agg_h(src_p, dst_p, g1, z_h)
    g2 = _t2(agg1[0], agg1[1], g1, dinv, b1, W2)
    z_h2 = jnp.zeros((CHUNK, HID2), jnp.float32)
    agg2 = _agg_h2(src_p, dst_p, g2, z_h2)
    logits = _t3(agg2[0], agg2[1], g2, dinv, b2, Wh.reshape(-1), bh)
    return logits[:N]


# speedup vs baseline: 29.8741x; 2.4825x over previous
import functools

import jax
import jax.numpy as jnp
from jax import lax
from jax.experimental import pallas as pl
from jax.experimental.pallas import tpu as pltpu
from jax.experimental.pallas import tpu_sc as plsc

N = 10000
E = 320000
IN_CH = 128
HIDDEN = 128
HID2 = 64

NC = 2
NS = 16
NW = NC * NS

N_PAD = 10240
CHUNK = 128
N_CHUNKS = 80
EDGES_PER_W = N_CHUNKS * CHUNK
E_PAD = NW * EDGES_PER_W
ROWS_PER_TILE = N_PAD // NS

_mesh = plsc.VectorSubcoreMesh(core_axis_name="c", subcore_axis_name="s")


@functools.partial(
    pl.kernel,
    mesh=_mesh,
    out_type=jax.ShapeDtypeStruct((NW, N_PAD), jnp.float32),
    scratch_types=[
        pltpu.VMEM((N_CHUNKS, CHUNK), jnp.int32),
        pltpu.VMEM((N_PAD,), jnp.float32),
    ],
    compiler_params=pltpu.CompilerParams(needs_layout_passes=False),
)
def _deg_kernel(dst_hbm, out_hbm, idx_v, dacc):
    c = lax.axis_index("c")
    s = lax.axis_index("s")
    wid = s * NC + c
    zero16 = jnp.zeros((16,), jnp.float32)
    one16 = jnp.ones((16,), jnp.float32)

    def zero_body(i, carry):
        dacc[pl.ds(i * 16, 16)] = zero16
        return carry

    lax.fori_loop(0, N_PAD // 16, zero_body, 0)

    pltpu.sync_copy(dst_hbm.at[pl.ds(wid * N_CHUNKS, N_CHUNKS)], idx_v)

    def inner(j, c2):
        idx = idx_v[j // (CHUNK // 16), pl.ds((j % (CHUNK // 16)) * 16, 16)]
        plsc.addupdate_scatter(dacc, [idx], one16)
        return c2

    lax.fori_loop(0, EDGES_PER_W // 16, inner, 0)
    pltpu.sync_copy(dacc, out_hbm.at[wid])


def _make_agg(D):
    seg = 40 if D >= 128 else N_CHUNKS
    nseg = N_CHUNKS // seg

    @functools.partial(
        pl.kernel,
        mesh=_mesh,
        out_type=jax.ShapeDtypeStruct((NC, N_PAD, D), jnp.float32),
        scratch_types=[
            pltpu.VMEM((seg, CHUNK), jnp.int32),
            pltpu.VMEM((N_CHUNKS, CHUNK), jnp.int32),
            pltpu.VMEM((CHUNK, D), jnp.float32),
            pltpu.VMEM((CHUNK, D), jnp.float32),
            pltpu.VMEM_SHARED((N_PAD, D), jnp.float32),
            pltpu.SemaphoreType.DMA,
            pltpu.SemaphoreType.DMA,
        ],
        compiler_params=pltpu.CompilerParams(
            use_tc_tiling_on_sc=(D % 128 == 0)),
    )
    def agg(src_hbm, dst_hbm, g_hbm, zeros_hbm, out_hbm,
            src_v, dst_v, rows_a, rows_b, acc, sem_a, sem_b):
        c = lax.axis_index("c")
        s = lax.axis_index("s")
        pltpu.sync_copy(zeros_hbm, rows_a)
        row0 = s * ROWS_PER_TILE
        for k in range(ROWS_PER_TILE // CHUNK):
            pltpu.sync_copy(rows_a, acc.at[pl.ds(row0 + k * CHUNK, CHUNK)])

        wid = c * NS + s
        pltpu.sync_copy(dst_hbm.at[pl.ds(wid * N_CHUNKS, N_CHUNKS)], dst_v)
        plsc.subcore_barrier()

        for sg in range(nseg):
            cbase = sg * seg
            pltpu.sync_copy(
                src_hbm.at[pl.ds(wid * N_CHUNKS + cbase, seg)], src_v)
            pltpu.async_copy(g_hbm.at[src_v.at[0]], rows_a, sem_a)

            def pipe_body(g, carry):
                i0 = 2 * g
                pltpu.async_copy(g_hbm.at[src_v.at[i0 + 1]], rows_b, sem_b)
                pltpu.make_async_copy(
                    g_hbm.at[src_v.at[i0]], rows_a, sem_a).wait()
                pltpu.sync_copy(rows_a, acc.at[dst_v.at[cbase + i0]],
                                add=True)

                @pl.when(g < seg // 2 - 1)
                def _():
                    pltpu.async_copy(g_hbm.at[src_v.at[i0 + 2]], rows_a,
                                     sem_a)

                pltpu.make_async_copy(
                    g_hbm.at[src_v.at[i0 + 1]], rows_b, sem_b).wait()
                pltpu.sync_copy(rows_b, acc.at[dst_v.at[cbase + i0 + 1]],
                                add=True)
                return carry

            lax.fori_loop(0, seg // 2, pipe_body, 0)

        plsc.subcore_barrier()
        for k in range(ROWS_PER_TILE // CHUNK):
            r = row0 + k * CHUNK
            pltpu.sync_copy(acc.at[pl.ds(r, CHUNK)], rows_a)
            pltpu.sync_copy(rows_a, out_hbm.at[c, pl.ds(r, CHUNK)])

    return agg


_agg_h = _make_agg(HIDDEN)
_agg_h2 = _make_agg(HID2)


BLK = 256


def _t1_body(x_ref, w1_ref, degs_ref, g_ref, dinv_ref):
    deg = jnp.sum(degs_ref[...], axis=0) + 1.0
    dinv = lax.rsqrt(deg)
    h = lax.dot_general(x_ref[...], w1_ref[...], (((1,), (1,)), ((), ())),
                        preferred_element_type=jnp.float32)
    g_ref[...] = h * dinv[:, None]
    dinv_ref[...] = dinv


_t1 = pl.pallas_call(
    _t1_body,
    grid=(N_PAD // BLK,),
    in_specs=[
        pl.BlockSpec((BLK, IN_CH), lambda i: (i, 0)),
        pl.BlockSpec((HIDDEN, IN_CH), lambda i: (0, 0)),
        pl.BlockSpec((NW, BLK), lambda i: (0, i)),
    ],
    out_specs=[
        pl.BlockSpec((BLK, HIDDEN), lambda i: (i, 0)),
        pl.BlockSpec((BLK,), lambda i: (i,)),
    ],
    out_shape=[
        jax.ShapeDtypeStruct((N_PAD, HIDDEN), jnp.float32),
        jax.ShapeDtypeStruct((N_PAD,), jnp.float32),
    ],
)


def _t2_body(a0_ref, a1_ref, g1_ref, dinv_ref, b1_ref, w2_ref, g2_ref):
    dinv = dinv_ref[...]
    h1 = (a0_ref[...] + a1_ref[...] + g1_ref[...]) * dinv[:, None]
    h1 = jnp.maximum(h1 + b1_ref[...][None, :], 0.0)
    h2 = lax.dot_general(h1, w2_ref[...], (((1,), (1,)), ((), ())),
                         preferred_element_type=jnp.float32)
    g2_ref[...] = h2 * dinv[:, None]


_t2 = pl.pallas_call(
    _t2_body,
    grid=(N_PAD // BLK,),
    in_specs=[
        pl.BlockSpec((BLK, HIDDEN), lambda i: (i, 0)),
        pl.BlockSpec((BLK, HIDDEN), lambda i: (i, 0)),
        pl.BlockSpec((BLK, HIDDEN), lambda i: (i, 0)),
        pl.BlockSpec((BLK,), lambda i: (i,)),
        pl.BlockSpec((HIDDEN,), lambda i: (0,)),
        pl.BlockSpec((HID2, HIDDEN), lambda i: (0, 0)),
    ],
    out_specs=pl.BlockSpec((BLK, HID2), lambda i: (i, 0)),
    out_shape=jax.ShapeDtypeStruct((N_PAD, HID2), jnp.float32),
)


def _t3_body(a0_ref, a1_ref, g2_ref, dinv_ref, b2_ref, wh_ref, bh_ref, out_ref):
    dinv = dinv_ref[...]
    h2 = (a0_ref[...] + a1_ref[...] + g2_ref[...]) * dinv[:, None]
    h2 = jnp.maximum(h2 + b2_ref[...][None, :], 0.0)
    out_ref[...] = jnp.sum(h2 * wh_ref[...][None, :], axis=1) + bh_ref[0]


_t3 = pl.pallas_call(
    _t3_body,
    grid=(N_PAD // BLK,),
    in_specs=[
        pl.BlockSpec((BLK, HID2), lambda i: (i, 0)),
        pl.BlockSpec((BLK, HID2), lambda i: (i, 0)),
        pl.BlockSpec((BLK, HID2), lambda i: (i, 0)),
        pl.BlockSpec((BLK,), lambda i: (i,)),
        pl.BlockSpec((HID2,), lambda i: (0,)),
        pl.BlockSpec((HID2,), lambda i: (0,)),
        pl.BlockSpec((1,), lambda i: (0,)),
    ],
    out_specs=pl.BlockSpec((BLK,), lambda i: (i,)),
    out_shape=jax.ShapeDtypeStruct((N_PAD,), jnp.float32),
)


def kernel(x, edge_index, W1, b1, W2, b2, Wh, bh):
    src = edge_index[0].astype(jnp.int32)
    dst = edge_index[1].astype(jnp.int32)
    dummy = N + (jnp.arange(E_PAD - E, dtype=jnp.int32) % (N_PAD - N))
    src_p = jnp.concatenate([src, dummy]).reshape(NW * N_CHUNKS, CHUNK)
    dst_p = jnp.concatenate([dst, dummy]).reshape(NW * N_CHUNKS, CHUNK)
    x_p = jnp.concatenate(
        [x, jnp.zeros((N_PAD - N, IN_CH), x.dtype)], axis=0)

    degs = _deg_kernel(dst_p)
    g1, dinv = _t1(x_p, W1, degs)
    z_h = jnp.zeros((CHUNK, HIDDEN), jnp.float32)
    agg1 = _agg_h(src_p, dst_p, g1, z_h)
    g2 = _t2(agg1[0], agg1[1], g1, dinv, b1, W2)
    z_h2 = jnp.zeros((CHUNK, HID2), jnp.float32)
    agg2 = _agg_h2(src_p, dst_p, g2, z_h2)
    logits = _t3(agg2[0], agg2[1], g2, dinv, b2, Wh.reshape(-1), bh)
    return logits[:N]
